# Initial kernel scaffold; baseline (speedup 1.0000x reference)
import functools

import jax
import jax.numpy as jnp
from jax import lax
from jax.experimental import pallas as pl
from jax.experimental.pallas import tpu as pltpu
from jax.experimental.pallas import tpu_sc as plsc

N = 10000
E = 320000
D = 128

NC = 2
NS = 16
NW = NC * NS
CHUNK = 128
NCHUNK = E // CHUNK
CPW = NCHUNK // NW
EXTRA = NCHUNK - CPW * NW
RPS = N // NS

_mesh = plsc.VectorSubcoreMesh(core_axis_name="c", subcore_axis_name="s")


def _sc_degree(dst2d, zeros16, ones16):
    @functools.partial(
        pl.kernel,
        mesh=_mesh,
        out_type=jax.ShapeDtypeStruct((NC, N, 16), jnp.float32),
        scratch_types=[
            pltpu.VMEM((1, CHUNK), jnp.int32),
            pltpu.VMEM((CHUNK, 16), jnp.float32),
            pltpu.VMEM_SHARED((N, 16), jnp.float32),
        ],
    )
    def k(dst_hbm, zeros_hbm, ones_hbm, out_hbm, idx_v, ones_v, acc):
        c = lax.axis_index("c")
        s = lax.axis_index("s")
        w = s * NC + c
        pltpu.sync_copy(ones_hbm, ones_v)
        pltpu.sync_copy(zeros_hbm.at[pl.ds(s * RPS, RPS)],
                        acc.at[pl.ds(s * RPS, RPS)])
        plsc.subcore_barrier()

        def do(chunk):
            pltpu.sync_copy(dst_hbm.at[pl.ds(chunk, 1)], idx_v)
            pltpu.sync_copy(ones_v, acc.at[idx_v.at[0]], add=True)

        @pl.loop(0, CPW)
        def _(j):
            do(j * NW + w)

        @pl.when(w < EXTRA)
        def _():
            do(CPW * NW + w)

        plsc.subcore_barrier()
        pltpu.sync_copy(acc.at[pl.ds(s * RPS, RPS)],
                        out_hbm.at[c, pl.ds(s * RPS, RPS)])

    return k(dst2d, zeros16, ones16)


def _sc_propagate(g, src2d, dst2d):
    @functools.partial(
        pl.kernel,
        mesh=_mesh,
        out_type=jax.ShapeDtypeStruct((NC, N, D), jnp.float32),
        scratch_types=[
            pltpu.VMEM((1, CHUNK), jnp.int32),
            pltpu.VMEM((1, CHUNK), jnp.int32),
            pltpu.VMEM((CHUNK, D), jnp.float32),
            pltpu.VMEM_SHARED((N, D), jnp.float32),
        ],
    )
    def k(g_hbm, src_hbm, dst_hbm, out_hbm, sidx, didx, rows, acc):
        c = lax.axis_index("c")
        s = lax.axis_index("s")
        w = s * NC + c
        pltpu.sync_copy(g_hbm.at[pl.ds(s * RPS, RPS)],
                        acc.at[pl.ds(s * RPS, RPS)])
        plsc.subcore_barrier()

        def do(chunk):
            pltpu.sync_copy(src_hbm.at[pl.ds(chunk, 1)], sidx)
            pltpu.sync_copy(dst_hbm.at[pl.ds(chunk, 1)], didx)
            pltpu.sync_copy(g_hbm.at[sidx.at[0]], rows)
            pltpu.sync_copy(rows, acc.at[didx.at[0]], add=True)

        @pl.loop(0, CPW)
        def _(j):
            do(j * NW + w)

        @pl.when(w < EXTRA)
        def _():
            do(CPW * NW + w)

        plsc.subcore_barrier()
        pltpu.sync_copy(acc.at[pl.ds(s * RPS, RPS)],
                        out_hbm.at[c, pl.ds(s * RPS, RPS)])

    return k(g, src2d, dst2d)


def _tc_mm(x, W):
    def body(x_ref, w_ref, o_ref):
        o_ref[...] = jnp.dot(x_ref[...], w_ref[...],
                             preferred_element_type=jnp.float32)

    return pl.pallas_call(
        body, out_shape=jax.ShapeDtypeStruct((x.shape[0], W.shape[1]),
                                             jnp.float32))(x, W)


def _tc_scale0(degp, h0):
    def body(degp_ref, h_ref, g_ref, dinvb_ref):
        deg = degp_ref[0] + degp_ref[1] + 1.0
        dinv = lax.rsqrt(deg[:, 0:1])
        dinvb = jnp.broadcast_to(dinv, (N, D))
        dinvb_ref[...] = dinvb
        g_ref[...] = dinvb * h_ref[...]

    return pl.pallas_call(
        body,
        out_shape=(jax.ShapeDtypeStruct((N, D), jnp.float32),
                   jax.ShapeDtypeStruct((N, D), jnp.float32)))(degp, h0)


def _tc_bn_relu_mm(zp, g, dinvb, b0, gamma0, beta0, W1):
    def body(zp_ref, g_ref, dinvb_ref, b_ref, gam_ref, bet_ref, w_ref, o_ref):
        dinvb = dinvb_ref[...]
        z = zp_ref[0] + zp_ref[1] - g_ref[...]
        y = dinvb * z + b_ref[...]
        mean = jnp.mean(y, axis=0, keepdims=True)
        var = jnp.mean((y - mean) ** 2, axis=0, keepdims=True)
        xn = (y - mean) * lax.rsqrt(var + 1e-5) * gam_ref[...] + bet_ref[...]
        xr = jnp.maximum(xn, 0.0)
        h1 = jnp.dot(xr, w_ref[...], preferred_element_type=jnp.float32)
        o_ref[...] = dinvb * h1

    return pl.pallas_call(
        body, out_shape=jax.ShapeDtypeStruct((N, D), jnp.float32))(
            zp, g, dinvb, b0, gamma0, beta0, W1)


def _tc_combine_mm(zp, g, dinvb, b1, Wf):
    def body(zp_ref, g_ref, dinvb_ref, b_ref, w_ref, o_ref):
        dinvb = dinvb_ref[...]
        z = zp_ref[0] + zp_ref[1] - g_ref[...]
        y = dinvb * z + b_ref[...]
        h2 = jnp.dot(y, w_ref[...], preferred_element_type=jnp.float32)
        o_ref[...] = dinvb * h2

    return pl.pallas_call(
        body, out_shape=jax.ShapeDtypeStruct((N, D), jnp.float32))(
            zp, g, dinvb, b1, Wf)


def _tc_final(zp, g, dinvb, bf):
    def body(zp_ref, g_ref, dinvb_ref, b_ref, o_ref):
        z = zp_ref[0] + zp_ref[1] - g_ref[...]
        o = dinvb_ref[...] * z + b_ref[...]
        m = jnp.max(o, axis=1, keepdims=True)
        lse = jnp.log(jnp.sum(jnp.exp(o - m), axis=1, keepdims=True)) + m
        o_ref[...] = o - lse

    return pl.pallas_call(
        body, out_shape=jax.ShapeDtypeStruct((N, D), jnp.float32))(
            zp, g, dinvb, bf)


def kernel(x, edge_index, W0, b0, gamma0, beta0, W1, b1, Wf, bf):
    ei = edge_index.astype(jnp.int32)
    src2d = ei[0].reshape(NCHUNK, CHUNK)
    dst2d = ei[1].reshape(NCHUNK, CHUNK)
    zeros16 = jnp.zeros((N, 16), jnp.float32)
    ones16 = jnp.ones((CHUNK, 16), jnp.float32)
    b0r = b0.reshape(1, D)
    gam = gamma0.reshape(1, D)
    bet = beta0.reshape(1, D)
    b1r = b1.reshape(1, D)
    bfr = bf.reshape(1, D)

    degp = _sc_degree(dst2d, zeros16, ones16)
    h0 = _tc_mm(x, W0)
    g0, dinvb = _tc_scale0(degp, h0)

    zp0 = _sc_propagate(g0, src2d, dst2d)
    g1 = _tc_bn_relu_mm(zp0, g0, dinvb, b0r, gam, bet, W1)

    zp1 = _sc_propagate(g1, src2d, dst2d)
    g2 = _tc_combine_mm(zp1, g1, dinvb, b1r, Wf)

    zp2 = _sc_propagate(g2, src2d, dst2d)
    return _tc_final(zp2, g2, dinvb, bfr)


# speedup vs baseline: 17.8354x; 17.8354x over previous
import functools

import jax
import jax.numpy as jnp
from jax import lax
from jax.experimental import pallas as pl
from jax.experimental.pallas import tpu as pltpu
from jax.experimental.pallas import tpu_sc as plsc

N = 10000
E = 320000
D = 128

NC = 2
NS = 16
NW = NC * NS
CHUNK = 125
NCHUNK = E // CHUNK
CPW = NCHUNK // NW
RMAIN = 624
RTAIL = N - RMAIN * NS

_mesh = plsc.VectorSubcoreMesh(core_axis_name="c", subcore_axis_name="s")


def _each_row_slice(s, fn):
    fn(s * RMAIN, RMAIN)

    @pl.when(s < RTAIL // 8)
    def _():
        fn(RMAIN * NS + s * 8, 8)


def _sc_degree(dst2d, zeros128, ones128):
    @functools.partial(
        pl.kernel,
        mesh=_mesh,
        out_type=jax.ShapeDtypeStruct((NC, N, D), jnp.float32),
        scratch_types=[
            pltpu.VMEM((CPW, CHUNK), jnp.int32),
            pltpu.VMEM((CHUNK, D), jnp.float32),
            pltpu.VMEM_SHARED((N, D), jnp.float32),
        ],
    )
    def k(dst_hbm, zeros_hbm, ones_hbm, out_hbm, idx_v, ones_v, acc):
        c = lax.axis_index("c")
        s = lax.axis_index("s")
        w = s * NC + c
        pltpu.sync_copy(ones_hbm, ones_v)
        pltpu.sync_copy(dst_hbm.at[pl.ds(w * CPW, CPW)], idx_v)
        _each_row_slice(s, lambda st, sz: pltpu.sync_copy(
            zeros_hbm.at[pl.ds(st, sz)], acc.at[pl.ds(st, sz)]))
        plsc.subcore_barrier()

        @pl.loop(0, CPW)
        def _(j):
            pltpu.sync_copy(ones_v, acc.at[idx_v.at[j]], add=True)

        plsc.subcore_barrier()
        _each_row_slice(s, lambda st, sz: pltpu.sync_copy(
            acc.at[pl.ds(st, sz)], out_hbm.at[c, pl.ds(st, sz)]))

    return k(dst2d, zeros128, ones128)


def _sc_propagate(g, src2d, dst2d):
    @functools.partial(
        pl.kernel,
        mesh=_mesh,
        out_type=jax.ShapeDtypeStruct((NC, N, D), jnp.float32),
        scratch_types=[
            pltpu.VMEM((CPW, CHUNK), jnp.int32),
            pltpu.VMEM((CPW, CHUNK), jnp.int32),
            pltpu.VMEM((CHUNK, D), jnp.float32),
            pltpu.VMEM_SHARED((N, D), jnp.float32),
        ],
    )
    def k(g_hbm, src_hbm, dst_hbm, out_hbm, sidx, didx, rows, acc):
        c = lax.axis_index("c")
        s = lax.axis_index("s")
        w = s * NC + c
        pltpu.sync_copy(src_hbm.at[pl.ds(w * CPW, CPW)], sidx)
        pltpu.sync_copy(dst_hbm.at[pl.ds(w * CPW, CPW)], didx)
        _each_row_slice(s, lambda st, sz: pltpu.sync_copy(
            g_hbm.at[pl.ds(st, sz)], acc.at[pl.ds(st, sz)]))
        plsc.subcore_barrier()

        @pl.loop(0, CPW)
        def _(j):
            pltpu.sync_copy(g_hbm.at[sidx.at[j]], rows)
            pltpu.sync_copy(rows, acc.at[didx.at[j]], add=True)

        plsc.subcore_barrier()
        _each_row_slice(s, lambda st, sz: pltpu.sync_copy(
            acc.at[pl.ds(st, sz)], out_hbm.at[c, pl.ds(st, sz)]))

    return k(g, src2d, dst2d)


def _tc_mm(x, W):
    def body(x_ref, w_ref, o_ref):
        o_ref[...] = jnp.dot(x_ref[...], w_ref[...],
                             preferred_element_type=jnp.float32)

    return pl.pallas_call(
        body, out_shape=jax.ShapeDtypeStruct((x.shape[0], W.shape[1]),
                                             jnp.float32))(x, W)


def _tc_scale0(degp, h0):
    def body(degp_ref, h_ref, g_ref, dinvb_ref):
        deg = degp_ref[0] + degp_ref[1] + 1.0
        dinvb = lax.rsqrt(deg)
        dinvb_ref[...] = dinvb
        g_ref[...] = dinvb * h_ref[...]

    return pl.pallas_call(
        body,
        out_shape=(jax.ShapeDtypeStruct((N, D), jnp.float32),
                   jax.ShapeDtypeStruct((N, D), jnp.float32)))(degp, h0)


def _tc_bn_relu_mm(zp, g, dinvb, b0, gamma0, beta0, W1):
    def body(zp_ref, g_ref, dinvb_ref, b_ref, gam_ref, bet_ref, w_ref, o_ref):
        dinvb = dinvb_ref[...]
        z = zp_ref[0] + zp_ref[1] - g_ref[...]
        y = dinvb * z + b_ref[...]
        mean = jnp.mean(y, axis=0, keepdims=True)
        var = jnp.mean((y - mean) ** 2, axis=0, keepdims=True)
        xn = (y - mean) * lax.rsqrt(var + 1e-5) * gam_ref[...] + bet_ref[...]
        xr = jnp.maximum(xn, 0.0)
        h1 = jnp.dot(xr, w_ref[...], preferred_element_type=jnp.float32)
        o_ref[...] = dinvb * h1

    return pl.pallas_call(
        body, out_shape=jax.ShapeDtypeStruct((N, D), jnp.float32))(
            zp, g, dinvb, b0, gamma0, beta0, W1)


def _tc_combine_mm(zp, g, dinvb, b1, Wf):
    def body(zp_ref, g_ref, dinvb_ref, b_ref, w_ref, o_ref):
        dinvb = dinvb_ref[...]
        z = zp_ref[0] + zp_ref[1] - g_ref[...]
        y = dinvb * z + b_ref[...]
        h2 = jnp.dot(y, w_ref[...], preferred_element_type=jnp.float32)
        o_ref[...] = dinvb * h2

    return pl.pallas_call(
        body, out_shape=jax.ShapeDtypeStruct((N, D), jnp.float32))(
            zp, g, dinvb, b1, Wf)


def _tc_final(zp, g, dinvb, bf):
    def body(zp_ref, g_ref, dinvb_ref, b_ref, o_ref):
        z = zp_ref[0] + zp_ref[1] - g_ref[...]
        o = dinvb_ref[...] * z + b_ref[...]
        m = jnp.max(o, axis=1, keepdims=True)
        lse = jnp.log(jnp.sum(jnp.exp(o - m), axis=1, keepdims=True)) + m
        o_ref[...] = o - lse

    return pl.pallas_call(
        body, out_shape=jax.ShapeDtypeStruct((N, D), jnp.float32))(
            zp, g, dinvb, bf)


def kernel(x, edge_index, W0, b0, gamma0, beta0, W1, b1, Wf, bf):
    ei = edge_index.astype(jnp.int32)
    src2d = ei[0].reshape(NCHUNK, CHUNK)
    dst2d = ei[1].reshape(NCHUNK, CHUNK)
    zeros128 = jnp.zeros((N, D), jnp.float32)
    ones128 = jnp.ones((CHUNK, D), jnp.float32)
    b0r = b0.reshape(1, D)
    gam = gamma0.reshape(1, D)
    bet = beta0.reshape(1, D)
    b1r = b1.reshape(1, D)
    bfr = bf.reshape(1, D)

    degp = _sc_degree(dst2d, zeros128, ones128)
    h0 = _tc_mm(x, W0)
    g0, dinvb = _tc_scale0(degp, h0)

    zp0 = _sc_propagate(g0, src2d, dst2d)
    g1 = _tc_bn_relu_mm(zp0, g0, dinvb, b0r, gam, bet, W1)

    zp1 = _sc_propagate(g1, src2d, dst2d)
    g2 = _tc_combine_mm(zp1, g1, dinvb, b1r, Wf)

    zp2 = _sc_propagate(g2, src2d, dst2d)
    return _tc_final(zp2, g2, dinvb, bfr)


# speedup vs baseline: 21.9735x; 1.2320x over previous
import functools

import jax
import jax.numpy as jnp
from jax import lax
from jax.experimental import pallas as pl
from jax.experimental.pallas import tpu as pltpu
from jax.experimental.pallas import tpu_sc as plsc

N = 10000
E = 320000
D = 128

NC = 2
NS = 16
NW = NC * NS
CHUNK = 125
NCHUNK = E // CHUNK
CPW = NCHUNK // NW
IQ = 16
RMAIN = 624
RTAIL = N - RMAIN * NS

_mesh = plsc.VectorSubcoreMesh(core_axis_name="c", subcore_axis_name="s")


def _each_row_slice(s, fn):
    fn(s * RMAIN, RMAIN)

    @pl.when(s < RTAIL // 8)
    def _():
        fn(RMAIN * NS + s * 8, 8)


def _sc_degree(dst2d, zeros128, ones128):
    @functools.partial(
        pl.kernel,
        mesh=_mesh,
        out_type=jax.ShapeDtypeStruct((NC, N, D), jnp.float32),
        scratch_types=[
            pltpu.VMEM((CPW, CHUNK), jnp.int32),
            pltpu.VMEM((CHUNK, D), jnp.float32),
            pltpu.VMEM_SHARED((N, D), jnp.float32),
            pltpu.SemaphoreType.DMA,
            pltpu.SemaphoreType.DMA,
        ],
    )
    def k(dst_hbm, zeros_hbm, ones_hbm, out_hbm, idx_v, ones_v, acc, sA, sB):
        c = lax.axis_index("c")
        s = lax.axis_index("s")
        w = s * NC + c
        pltpu.sync_copy(ones_hbm, ones_v)
        pltpu.sync_copy(dst_hbm.at[pl.ds(w * CPW, CPW)], idx_v)
        _each_row_slice(s, lambda st, sz: pltpu.sync_copy(
            zeros_hbm.at[pl.ds(st, sz)], acc.at[pl.ds(st, sz)]))
        plsc.subcore_barrier()

        @pl.loop(0, CPW)
        def _(j):
            pltpu.sync_copy(ones_v, acc.at[idx_v.at[j]], add=True)

        plsc.subcore_barrier()
        _each_row_slice(s, lambda st, sz: pltpu.sync_copy(
            acc.at[pl.ds(st, sz)], out_hbm.at[c, pl.ds(st, sz)]))

    return k(dst2d, zeros128, ones128)


def _sc_propagate(g, src2d, dst2d):
    @functools.partial(
        pl.kernel,
        mesh=_mesh,
        out_type=jax.ShapeDtypeStruct((NC, N, D), jnp.float32),
        scratch_types=[
            pltpu.VMEM((IQ, CHUNK), jnp.int32),
            pltpu.VMEM((IQ, CHUNK), jnp.int32),
            pltpu.VMEM((2, CHUNK, D), jnp.float32),
            pltpu.VMEM_SHARED((N, D), jnp.float32),
            pltpu.SemaphoreType.DMA((2,)),
        ],
    )
    def k(g_hbm, src_hbm, dst_hbm, out_hbm, sidx, didx, rows, acc, gsem):
        c = lax.axis_index("c")
        s = lax.axis_index("s")
        w = s * NC + c
        pltpu.sync_copy(src_hbm.at[pl.ds(w * CPW, IQ)], sidx)
        pltpu.sync_copy(dst_hbm.at[pl.ds(w * CPW, IQ)], didx)
        _each_row_slice(s, lambda st, sz: pltpu.sync_copy(
            g_hbm.at[pl.ds(st, sz)], acc.at[pl.ds(st, sz)]))
        plsc.subcore_barrier()

        def g_start(j):
            pltpu.async_copy(g_hbm.at[sidx.at[j % IQ]], rows.at[j % 2],
                             gsem.at[j % 2])

        def g_wait(j):
            pltpu.make_async_copy(g_hbm.at[sidx.at[j % IQ]], rows.at[j % 2],
                                  gsem.at[j % 2]).wait()

        g_start(0)

        @pl.loop(0, CPW)
        def _(j):
            g_wait(j)

            @pl.when(j < CPW - 1)
            def _():
                jj = j + 1
                @pl.when(jj % IQ == 0)
                def _():
                    off = pl.multiple_of(w * CPW + jj, 8)
                    pltpu.sync_copy(src_hbm.at[pl.ds(off, IQ)], sidx)

                g_start(jj)

            @pl.when(jnp.logical_and(j % IQ == 0, j > 0))
            def _():
                off = pl.multiple_of(w * CPW + j, 8)
                pltpu.sync_copy(dst_hbm.at[pl.ds(off, IQ)], didx)

            pltpu.sync_copy(rows.at[j % 2], acc.at[didx.at[j % IQ]], add=True)

        plsc.subcore_barrier()
        _each_row_slice(s, lambda st, sz: pltpu.sync_copy(
            acc.at[pl.ds(st, sz)], out_hbm.at[c, pl.ds(st, sz)]))

    return k(g, src2d, dst2d)


def _tc_mm(x, W):
    def body(x_ref, w_ref, o_ref):
        o_ref[...] = jnp.dot(x_ref[...], w_ref[...],
                             preferred_element_type=jnp.float32)

    return pl.pallas_call(
        body, out_shape=jax.ShapeDtypeStruct((x.shape[0], W.shape[1]),
                                             jnp.float32))(x, W)


def _tc_scale0(degp, h0):
    def body(degp_ref, h_ref, g_ref, dinvb_ref):
        deg = degp_ref[0] + degp_ref[1] + 1.0
        dinvb = lax.rsqrt(deg)
        dinvb_ref[...] = dinvb
        g_ref[...] = dinvb * h_ref[...]

    return pl.pallas_call(
        body,
        out_shape=(jax.ShapeDtypeStruct((N, D), jnp.float32),
                   jax.ShapeDtypeStruct((N, D), jnp.float32)))(degp, h0)


def _tc_bn_relu_mm(zp, g, dinvb, b0, gamma0, beta0, W1):
    def body(zp_ref, g_ref, dinvb_ref, b_ref, gam_ref, bet_ref, w_ref, o_ref):
        dinvb = dinvb_ref[...]
        z = zp_ref[0] + zp_ref[1] - g_ref[...]
        y = dinvb * z + b_ref[...]
        mean = jnp.mean(y, axis=0, keepdims=True)
        var = jnp.mean((y - mean) ** 2, axis=0, keepdims=True)
        xn = (y - mean) * lax.rsqrt(var + 1e-5) * gam_ref[...] + bet_ref[...]
        xr = jnp.maximum(xn, 0.0)
        h1 = jnp.dot(xr, w_ref[...], preferred_element_type=jnp.float32)
        o_ref[...] = dinvb * h1

    return pl.pallas_call(
        body, out_shape=jax.ShapeDtypeStruct((N, D), jnp.float32))(
            zp, g, dinvb, b0, gamma0, beta0, W1)


def _tc_combine_mm(zp, g, dinvb, b1, Wf):
    def body(zp_ref, g_ref, dinvb_ref, b_ref, w_ref, o_ref):
        dinvb = dinvb_ref[...]
        z = zp_ref[0] + zp_ref[1] - g_ref[...]
        y = dinvb * z + b_ref[...]
        h2 = jnp.dot(y, w_ref[...], preferred_element_type=jnp.float32)
        o_ref[...] = dinvb * h2

    return pl.pallas_call(
        body, out_shape=jax.ShapeDtypeStruct((N, D), jnp.float32))(
            zp, g, dinvb, b1, Wf)


def _tc_final(zp, g, dinvb, bf):
    def body(zp_ref, g_ref, dinvb_ref, b_ref, o_ref):
        z = zp_ref[0] + zp_ref[1] - g_ref[...]
        o = dinvb_ref[...] * z + b_ref[...]
        m = jnp.max(o, axis=1, keepdims=True)
        lse = jnp.log(jnp.sum(jnp.exp(o - m), axis=1, keepdims=True)) + m
        o_ref[...] = o - lse

    return pl.pallas_call(
        body, out_shape=jax.ShapeDtypeStruct((N, D), jnp.float32))(
            zp, g, dinvb, bf)


def kernel(x, edge_index, W0, b0, gamma0, beta0, W1, b1, Wf, bf):
    ei = edge_index.astype(jnp.int32)
    src2d = ei[0].reshape(NCHUNK, CHUNK)
    dst2d = ei[1].reshape(NCHUNK, CHUNK)
    zeros128 = jnp.zeros((N, D), jnp.float32)
    ones128 = jnp.ones((CHUNK, D), jnp.float32)
    b0r = b0.reshape(1, D)
    gam = gamma0.reshape(1, D)
    bet = beta0.reshape(1, D)
    b1r = b1.reshape(1, D)
    bfr = bf.reshape(1, D)

    degp = _sc_degree(dst2d, zeros128, ones128)
    h0 = _tc_mm(x, W0)
    g0, dinvb = _tc_scale0(degp, h0)

    zp0 = _sc_propagate(g0, src2d, dst2d)
    g1 = _tc_bn_relu_mm(zp0, g0, dinvb, b0r, gam, bet, W1)

    zp1 = _sc_propagate(g1, src2d, dst2d)
    g2 = _tc_combine_mm(zp1, g1, dinvb, b1r, Wf)

    zp2 = _sc_propagate(g2, src2d, dst2d)
    return _tc_final(zp2, g2, dinvb, bfr)


# speedup vs baseline: 22.0838x; 1.0050x over previous
import functools

import jax
import jax.numpy as jnp
from jax import lax
from jax.experimental import pallas as pl
from jax.experimental.pallas import tpu as pltpu
from jax.experimental.pallas import tpu_sc as plsc

N = 10000
E = 320000
D = 128

NC = 2
NS = 16
NW = NC * NS
CHUNK = 125
NCHUNK = E // CHUNK
CPW = NCHUNK // NW
IQ = 16
RMAIN = 624
RTAIL = N - RMAIN * NS

_mesh = plsc.VectorSubcoreMesh(core_axis_name="c", subcore_axis_name="s")


def _each_row_slice(s, fn):
    fn(s * RMAIN, RMAIN)

    @pl.when(s < RTAIL // 8)
    def _():
        fn(RMAIN * NS + s * 8, 8)


def _sc_degree(dst2d, zeros128, ones128):
    @functools.partial(
        pl.kernel,
        mesh=_mesh,
        out_type=jax.ShapeDtypeStruct((NC, N, D), jnp.float32),
        scratch_types=[
            pltpu.VMEM((CPW, CHUNK), jnp.int32),
            pltpu.VMEM((CHUNK, D), jnp.float32),
            pltpu.VMEM_SHARED((N, D), jnp.float32),
            pltpu.SemaphoreType.DMA((2,)),
        ],
    )
    def k(dst_hbm, zeros_hbm, ones_hbm, out_hbm, idx_v, ones_v, acc, sem):
        c = lax.axis_index("c")
        s = lax.axis_index("s")
        w = s * NC + c
        pltpu.sync_copy(ones_hbm, ones_v)
        pltpu.sync_copy(dst_hbm.at[pl.ds(w * CPW, CPW)], idx_v)
        _each_row_slice(s, lambda st, sz: pltpu.sync_copy(
            zeros_hbm.at[pl.ds(st, sz)], acc.at[pl.ds(st, sz)]))
        plsc.subcore_barrier()

        def s_start(j):
            pltpu.async_copy(ones_v, acc.at[idx_v.at[j]], sem.at[j % 2],
                             add=True)

        def s_wait(j):
            pltpu.make_async_copy(ones_v, acc.at[idx_v.at[j]],
                                  sem.at[j % 2]).wait()

        @pl.loop(0, CPW)
        def _(j):
            @pl.when(j >= 2)
            def _():
                s_wait(j - 2)

            s_start(j)

        s_wait(CPW - 2)
        s_wait(CPW - 1)
        plsc.subcore_barrier()
        _each_row_slice(s, lambda st, sz: pltpu.sync_copy(
            acc.at[pl.ds(st, sz)], out_hbm.at[c, pl.ds(st, sz)]))

    return k(dst2d, zeros128, ones128)


def _sc_propagate(g, src2d, dst2d):
    @functools.partial(
        pl.kernel,
        mesh=_mesh,
        out_type=jax.ShapeDtypeStruct((NC, N, D), jnp.float32),
        scratch_types=[
            pltpu.VMEM((IQ, CHUNK), jnp.int32),
            pltpu.VMEM((IQ, CHUNK), jnp.int32),
            pltpu.VMEM((2, CHUNK, D), jnp.float32),
            pltpu.VMEM_SHARED((N, D), jnp.float32),
            pltpu.SemaphoreType.DMA((2,)),
            pltpu.SemaphoreType.DMA((2,)),
        ],
    )
    def k(g_hbm, src_hbm, dst_hbm, out_hbm, sidx, didx, rows, acc, gsem, ssem):
        c = lax.axis_index("c")
        s = lax.axis_index("s")
        w = s * NC + c
        pltpu.sync_copy(src_hbm.at[pl.ds(w * CPW, IQ)], sidx)
        pltpu.sync_copy(dst_hbm.at[pl.ds(w * CPW, IQ)], didx)
        _each_row_slice(s, lambda st, sz: pltpu.sync_copy(
            g_hbm.at[pl.ds(st, sz)], acc.at[pl.ds(st, sz)]))
        plsc.subcore_barrier()

        def g_start(j):
            pltpu.async_copy(g_hbm.at[sidx.at[j % IQ]], rows.at[j % 2],
                             gsem.at[j % 2])

        def g_wait(j):
            pltpu.make_async_copy(g_hbm.at[sidx.at[j % IQ]], rows.at[j % 2],
                                  gsem.at[j % 2]).wait()

        def s_start(j):
            pltpu.async_copy(rows.at[j % 2], acc.at[didx.at[j % IQ]],
                             ssem.at[j % 2], add=True)

        def s_wait(j):
            pltpu.make_async_copy(rows.at[j % 2], acc.at[didx.at[j % IQ]],
                                  ssem.at[j % 2]).wait()

        g_start(0)

        @pl.loop(0, CPW)
        def _(j):
            g_wait(j)

            @pl.when(j >= 1)
            def _():
                s_wait(j - 1)

            @pl.when(j < CPW - 1)
            def _():
                jj = j + 1
                @pl.when(jj % IQ == 0)
                def _():
                    off = pl.multiple_of(w * CPW + jj, 8)
                    pltpu.sync_copy(src_hbm.at[pl.ds(off, IQ)], sidx)

                g_start(jj)

            @pl.when(jnp.logical_and(j % IQ == 0, j > 0))
            def _():
                off = pl.multiple_of(w * CPW + j, 8)
                pltpu.sync_copy(dst_hbm.at[pl.ds(off, IQ)], didx)

            s_start(j)

        s_wait(CPW - 1)
        plsc.subcore_barrier()
        _each_row_slice(s, lambda st, sz: pltpu.sync_copy(
            acc.at[pl.ds(st, sz)], out_hbm.at[c, pl.ds(st, sz)]))

    return k(g, src2d, dst2d)


def _tc_mm(x, W):
    def body(x_ref, w_ref, o_ref):
        o_ref[...] = jnp.dot(x_ref[...], w_ref[...],
                             preferred_element_type=jnp.float32)

    return pl.pallas_call(
        body, out_shape=jax.ShapeDtypeStruct((x.shape[0], W.shape[1]),
                                             jnp.float32))(x, W)


def _tc_scale0(degp, h0):
    def body(degp_ref, h_ref, g_ref, dinvb_ref):
        deg = degp_ref[0] + degp_ref[1] + 1.0
        dinvb = lax.rsqrt(deg)
        dinvb_ref[...] = dinvb
        g_ref[...] = dinvb * h_ref[...]

    return pl.pallas_call(
        body,
        out_shape=(jax.ShapeDtypeStruct((N, D), jnp.float32),
                   jax.ShapeDtypeStruct((N, D), jnp.float32)))(degp, h0)


def _tc_bn_relu_mm(zp, g, dinvb, b0, gamma0, beta0, W1):
    def body(zp_ref, g_ref, dinvb_ref, b_ref, gam_ref, bet_ref, w_ref, o_ref):
        dinvb = dinvb_ref[...]
        z = zp_ref[0] + zp_ref[1] - g_ref[...]
        y = dinvb * z + b_ref[...]
        mean = jnp.mean(y, axis=0, keepdims=True)
        var = jnp.mean((y - mean) ** 2, axis=0, keepdims=True)
        xn = (y - mean) * lax.rsqrt(var + 1e-5) * gam_ref[...] + bet_ref[...]
        xr = jnp.maximum(xn, 0.0)
        h1 = jnp.dot(xr, w_ref[...], preferred_element_type=jnp.float32)
        o_ref[...] = dinvb * h1

    return pl.pallas_call(
        body, out_shape=jax.ShapeDtypeStruct((N, D), jnp.float32))(
            zp, g, dinvb, b0, gamma0, beta0, W1)


def _tc_combine_mm(zp, g, dinvb, b1, Wf):
    def body(zp_ref, g_ref, dinvb_ref, b_ref, w_ref, o_ref):
        dinvb = dinvb_ref[...]
        z = zp_ref[0] + zp_ref[1] - g_ref[...]
        y = dinvb * z + b_ref[...]
        h2 = jnp.dot(y, w_ref[...], preferred_element_type=jnp.float32)
        o_ref[...] = dinvb * h2

    return pl.pallas_call(
        body, out_shape=jax.ShapeDtypeStruct((N, D), jnp.float32))(
            zp, g, dinvb, b1, Wf)


def _tc_final(zp, g, dinvb, bf):
    def body(zp_ref, g_ref, dinvb_ref, b_ref, o_ref):
        z = zp_ref[0] + zp_ref[1] - g_ref[...]
        o = dinvb_ref[...] * z + b_ref[...]
        m = jnp.max(o, axis=1, keepdims=True)
        lse = jnp.log(jnp.sum(jnp.exp(o - m), axis=1, keepdims=True)) + m
        o_ref[...] = o - lse

    return pl.pallas_call(
        body, out_shape=jax.ShapeDtypeStruct((N, D), jnp.float32))(
            zp, g, dinvb, bf)


def kernel(x, edge_index, W0, b0, gamma0, beta0, W1, b1, Wf, bf):
    ei = edge_index.astype(jnp.int32)
    src2d = ei[0].reshape(NCHUNK, CHUNK)
    dst2d = ei[1].reshape(NCHUNK, CHUNK)
    zeros128 = jnp.zeros((N, D), jnp.float32)
    ones128 = jnp.ones((CHUNK, D), jnp.float32)
    b0r = b0.reshape(1, D)
    gam = gamma0.reshape(1, D)
    bet = beta0.reshape(1, D)
    b1r = b1.reshape(1, D)
    bfr = bf.reshape(1, D)

    degp = _sc_degree(dst2d, zeros128, ones128)
    h0 = _tc_mm(x, W0)
    g0, dinvb = _tc_scale0(degp, h0)

    zp0 = _sc_propagate(g0, src2d, dst2d)
    g1 = _tc_bn_relu_mm(zp0, g0, dinvb, b0r, gam, bet, W1)

    zp1 = _sc_propagate(g1, src2d, dst2d)
    g2 = _tc_combine_mm(zp1, g1, dinvb, b1r, Wf)

    zp2 = _sc_propagate(g2, src2d, dst2d)
    return _tc_final(zp2, g2, dinvb, bfr)


# speedup vs baseline: 24.7448x; 1.1205x over previous
import dataclasses
import functools

import jax
import jax.numpy as jnp
from jax import lax
from jax.experimental import pallas as pl
from jax.experimental.pallas import tpu as pltpu
from jax.experimental.pallas import tpu_sc as plsc

N = 10000
E = 320000
D = 128

NC = 2
NS = 16
NW = NC * NS
CHUNK = 125
NCHUNK = E // CHUNK
CPW = NCHUNK // NW
IQ = 16
RMAIN = 624
RTAIL = N - RMAIN * NS

_mesh = plsc.VectorSubcoreMesh(core_axis_name="c", subcore_axis_name="s")

_cp_no_layout = pltpu.CompilerParams()
if "needs_layout_passes" in pltpu.CompilerParams.__dataclass_fields__:
    _cp_no_layout = dataclasses.replace(_cp_no_layout,
                                        needs_layout_passes=False)


def _each_row_slice(s, fn):
    fn(s * RMAIN, RMAIN)

    @pl.when(s < RTAIL // 8)
    def _():
        fn(RMAIN * NS + s * 8, 8)


DROWS = 80
EPW = E // NW
NVEC = EPW // 16


def _sc_degree(dst1d):
    @functools.partial(
        pl.kernel,
        mesh=_mesh,
        out_type=jax.ShapeDtypeStruct((NC, DROWS, D), jnp.float32),
        compiler_params=_cp_no_layout,
        scratch_types=[
            pltpu.VMEM((EPW,), jnp.int32),
            pltpu.VMEM((8, DROWS, D), jnp.float32),
            pltpu.VMEM((1, DROWS), jnp.int32),
            pltpu.VMEM_SHARED((DROWS, D), jnp.float32),
        ],
    )
    def k(dst_hbm, out_hbm, idx_v, hist, rowid, accd):
        c = lax.axis_index("c")
        s = lax.axis_index("s")
        w = s * NC + c
        pltpu.sync_copy(dst_hbm.at[pl.ds(w * EPW, EPW)], idx_v)

        zeros16 = jnp.zeros((16,), jnp.float32)
        iota = lax.iota(jnp.int32, 16)

        @pl.loop(0, 8)
        def _(p):
            @pl.loop(0, DROWS)
            def _(r):
                @pl.loop(0, D // 16)
                def _(cc):
                    hist[p, r, pl.ds(cc * 16, 16)] = zeros16

        @pl.loop(0, DROWS // 16)
        def _(kk):
            rowid[0, pl.ds(kk * 16, 16)] = iota + kk * 16

        @pl.when(s == 0)
        def _():
            pltpu.sync_copy(hist.at[0], accd)

        plane = iota & 7
        mask_lo = iota < 8
        mask_hi = iota >= 8
        ones16 = jnp.ones((16,), jnp.float32)

        @pl.loop(0, NVEC)
        def _(v):
            d = idx_v[pl.ds(v * 16, 16)]
            r = lax.shift_right_logical(d, 7)
            col = d & 127
            plsc.addupdate_scatter(hist, [plane, r, col], ones16, mask=mask_lo)
            plsc.addupdate_scatter(hist, [plane, r, col], ones16, mask=mask_hi)

        plsc.subcore_barrier()

        @pl.loop(0, 8)
        def _(p):
            pltpu.sync_copy(hist.at[p], accd.at[rowid.at[0]], add=True)

        plsc.subcore_barrier()

        @pl.when(s < DROWS // 8)
        def _():
            pltpu.sync_copy(accd.at[pl.ds(s * 8, 8)],
                            out_hbm.at[c, pl.ds(s * 8, 8)])

    return k(dst1d)


def _sc_propagate(g, src2d, dst2d):
    @functools.partial(
        pl.kernel,
        mesh=_mesh,
        out_type=jax.ShapeDtypeStruct((NC, N, D), jnp.float32),
        scratch_types=[
            pltpu.VMEM((IQ, CHUNK), jnp.int32),
            pltpu.VMEM((IQ, CHUNK), jnp.int32),
            pltpu.VMEM((2, CHUNK, D), jnp.float32),
            pltpu.VMEM_SHARED((N, D), jnp.float32),
            pltpu.SemaphoreType.DMA((2,)),
            pltpu.SemaphoreType.DMA((2,)),
        ],
    )
    def k(g_hbm, src_hbm, dst_hbm, out_hbm, sidx, didx, rows, acc, gsem, ssem):
        c = lax.axis_index("c")
        s = lax.axis_index("s")
        w = s * NC + c
        pltpu.sync_copy(src_hbm.at[pl.ds(w * CPW, IQ)], sidx)
        pltpu.sync_copy(dst_hbm.at[pl.ds(w * CPW, IQ)], didx)
        _each_row_slice(s, lambda st, sz: pltpu.sync_copy(
            g_hbm.at[pl.ds(st, sz)], acc.at[pl.ds(st, sz)]))
        plsc.subcore_barrier()

        def g_start(j):
            pltpu.async_copy(g_hbm.at[sidx.at[j % IQ]], rows.at[j % 2],
                             gsem.at[j % 2])

        def g_wait(j):
            pltpu.make_async_copy(g_hbm.at[sidx.at[j % IQ]], rows.at[j % 2],
                                  gsem.at[j % 2]).wait()

        def s_start(j):
            pltpu.async_copy(rows.at[j % 2], acc.at[didx.at[j % IQ]],
                             ssem.at[j % 2], add=True)

        def s_wait(j):
            pltpu.make_async_copy(rows.at[j % 2], acc.at[didx.at[j % IQ]],
                                  ssem.at[j % 2]).wait()

        g_start(0)

        @pl.loop(0, CPW)
        def _(j):
            g_wait(j)

            @pl.when(j >= 1)
            def _():
                s_wait(j - 1)

            @pl.when(j < CPW - 1)
            def _():
                jj = j + 1
                @pl.when(jj % IQ == 0)
                def _():
                    off = pl.multiple_of(w * CPW + jj, 8)
                    pltpu.sync_copy(src_hbm.at[pl.ds(off, IQ)], sidx)

                g_start(jj)

            @pl.when(jnp.logical_and(j % IQ == 0, j > 0))
            def _():
                off = pl.multiple_of(w * CPW + j, 8)
                pltpu.sync_copy(dst_hbm.at[pl.ds(off, IQ)], didx)

            s_start(j)

        s_wait(CPW - 1)
        plsc.subcore_barrier()
        _each_row_slice(s, lambda st, sz: pltpu.sync_copy(
            acc.at[pl.ds(st, sz)], out_hbm.at[c, pl.ds(st, sz)]))

    return k(g, src2d, dst2d)


def _tc_mm_scale0(x, W0, degp2):
    def body(x_ref, w_ref, degp_ref, g_ref, dinvb_ref):
        deg = degp_ref[0] + degp_ref[1] + 1.0
        dinvb = jnp.broadcast_to(lax.rsqrt(deg), (N, D))
        dinvb_ref[...] = dinvb
        h0 = jnp.dot(x_ref[...], w_ref[...],
                     preferred_element_type=jnp.float32)
        g_ref[...] = dinvb * h0

    return pl.pallas_call(
        body,
        out_shape=(jax.ShapeDtypeStruct((N, D), jnp.float32),
                   jax.ShapeDtypeStruct((N, D), jnp.float32)))(x, W0, degp2)


def _tc_bn_relu_mm(zp, g, dinvb, b0, gamma0, beta0, W1):
    def body(zp_ref, g_ref, dinvb_ref, b_ref, gam_ref, bet_ref, w_ref, o_ref):
        dinvb = dinvb_ref[...]
        z = zp_ref[0] + zp_ref[1] - g_ref[...]
        y = dinvb * z + b_ref[...]
        mean = jnp.mean(y, axis=0, keepdims=True)
        var = jnp.mean((y - mean) ** 2, axis=0, keepdims=True)
        xn = (y - mean) * lax.rsqrt(var + 1e-5) * gam_ref[...] + bet_ref[...]
        xr = jnp.maximum(xn, 0.0)
        h1 = jnp.dot(xr, w_ref[...], preferred_element_type=jnp.float32)
        o_ref[...] = dinvb * h1

    return pl.pallas_call(
        body, out_shape=jax.ShapeDtypeStruct((N, D), jnp.float32))(
            zp, g, dinvb, b0, gamma0, beta0, W1)


def _tc_combine_mm(zp, g, dinvb, b1, Wf):
    def body(zp_ref, g_ref, dinvb_ref, b_ref, w_ref, o_ref):
        dinvb = dinvb_ref[...]
        z = zp_ref[0] + zp_ref[1] - g_ref[...]
        y = dinvb * z + b_ref[...]
        h2 = jnp.dot(y, w_ref[...], preferred_element_type=jnp.float32)
        o_ref[...] = dinvb * h2

    return pl.pallas_call(
        body, out_shape=jax.ShapeDtypeStruct((N, D), jnp.float32))(
            zp, g, dinvb, b1, Wf)


def _tc_final(zp, g, dinvb, bf):
    def body(zp_ref, g_ref, dinvb_ref, b_ref, o_ref):
        z = zp_ref[0] + zp_ref[1] - g_ref[...]
        o = dinvb_ref[...] * z + b_ref[...]
        m = jnp.max(o, axis=1, keepdims=True)
        lse = jnp.log(jnp.sum(jnp.exp(o - m), axis=1, keepdims=True)) + m
        o_ref[...] = o - lse

    return pl.pallas_call(
        body, out_shape=jax.ShapeDtypeStruct((N, D), jnp.float32))(
            zp, g, dinvb, bf)


def kernel(x, edge_index, W0, b0, gamma0, beta0, W1, b1, Wf, bf):
    ei = edge_index.astype(jnp.int32)
    src2d = ei[0].reshape(NCHUNK, CHUNK)
    dst2d = ei[1].reshape(NCHUNK, CHUNK)
    b0r = b0.reshape(1, D)
    gam = gamma0.reshape(1, D)
    bet = beta0.reshape(1, D)
    b1r = b1.reshape(1, D)
    bfr = bf.reshape(1, D)

    degp = _sc_degree(ei[1])
    degp2 = degp.reshape(NC, DROWS * D)[:, :N, None]
    g0, dinvb = _tc_mm_scale0(x, W0, degp2)

    zp0 = _sc_propagate(g0, src2d, dst2d)
    g1 = _tc_bn_relu_mm(zp0, g0, dinvb, b0r, gam, bet, W1)

    zp1 = _sc_propagate(g1, src2d, dst2d)
    g2 = _tc_combine_mm(zp1, g1, dinvb, b1r, Wf)

    zp2 = _sc_propagate(g2, src2d, dst2d)
    return _tc_final(zp2, g2, dinvb, bfr)
